# Initial kernel scaffold; baseline (speedup 1.0000x reference)
import jax
import jax.numpy as jnp
from jax.experimental import pallas as pl


def kernel(h, y, num_edges):
    raise NotImplementedError("write your pallas kernel here")


# speedup vs baseline: 1.5709x; 1.5709x over previous
import jax
import jax.numpy as jnp
from jax.experimental import pallas as pl
from jax.experimental.pallas import tpu as pltpu


_B = 2000


def _body(scale_ref, h_ref, y_ref, o_ref):
    x = h_ref[...]
    yv = y_ref[...]
    b, c = x.shape
    m = jnp.max(x, axis=-1, keepdims=True)
    s = jnp.sum(jnp.exp(x - m), axis=-1)
    lse = m[:, 0] + jnp.log(s)
    col = jax.lax.broadcasted_iota(jnp.int32, (b, c), 1)
    sel = jnp.sum(jnp.where(col == yv[0, 0][:, None], x, 0.0), axis=-1)
    o_ref[0, 0, :] = (sel - lse) * scale_ref[0]


def kernel(h, y, num_edges):
    n, c = h.shape
    nb = n // _B
    scale = (num_edges / n).astype(jnp.float32).reshape(1)
    y3 = y.astype(jnp.int32).reshape(nb, 1, _B)
    out = pl.pallas_call(
        _body,
        grid=(nb,),
        in_specs=[
            pl.BlockSpec(memory_space=pltpu.SMEM),
            pl.BlockSpec((_B, c), lambda i: (i, 0)),
            pl.BlockSpec((1, 1, _B), lambda i: (i, 0, 0)),
        ],
        out_specs=pl.BlockSpec((1, 1, _B), lambda i: (i, 0, 0)),
        out_shape=jax.ShapeDtypeStruct((nb, 1, _B), jnp.float32),
    )(scale, h, y3)
    return out.reshape(n)
